# Initial kernel scaffold; baseline (speedup 1.0000x reference)
import functools

import jax
import jax.numpy as jnp
from jax import lax
from jax.experimental import pallas as pl
from jax.experimental.pallas import tpu as pltpu
from jax.experimental.pallas import tpu_sc as plsc

LANES = 16
SUB = 128
NSUB = 4
CHUNK = SUB * NSUB


@functools.cache
def _build(n_total, num_tables, vocab, d):
    info = plsc.get_sparse_core_info()
    nc, ns = info.num_cores, info.num_subcores
    nw = nc * ns
    per_tile = n_total // nw
    n_chunks = per_tile // CHUNK
    out_chunk = CHUNK // num_tables
    out_per_tile = per_tile // num_tables
    n_out = n_total // num_tables
    assert n_total % (nw * CHUNK) == 0 and CHUNK % num_tables == 0
    assert d % LANES == 0 and LANES % num_tables == 0 and n_chunks % 2 == 0

    mesh = plsc.VectorSubcoreMesh(core_axis_name="c", subcore_axis_name="s")

    @functools.partial(
        pl.kernel,
        mesh=mesh,
        out_type=jax.ShapeDtypeStruct((n_out, d), jnp.float32),
        scratch_types=[
            pltpu.VMEM((per_tile,), jnp.int32),
            pltpu.VMEM((2, NSUB, SUB), jnp.int32),
            pltpu.VMEM((2, CHUNK, d), jnp.float32),
            pltpu.VMEM((2, out_chunk, d), jnp.float32),
            pltpu.SemaphoreType.DMA,
            pltpu.SemaphoreType.DMA,
            pltpu.SemaphoreType.DMA,
            pltpu.SemaphoreType.DMA,
        ],
    )
    def k(ids_hbm, table_hbm, out_hbm, ids_v, idx_v, rows_v, out_v,
          gsem0, gsem1, osem0, osem1):
        wid = lax.axis_index("s") * nc + lax.axis_index("c")
        base = wid * per_tile
        out_base = wid * out_per_tile
        gsems = (gsem0, gsem1)
        osems = (osem0, osem1)

        pltpu.sync_copy(ids_hbm.at[pl.ds(base, per_tile)], ids_v)
        offs = (lax.iota(jnp.int32, LANES) % num_tables) * vocab

        def fire(c, b):
            co = pl.multiple_of(c * CHUNK, CHUNK)
            for s in range(NSUB):
                for v in range(SUB // LANES):
                    idx_v[b, s, pl.ds(v * LANES, LANES)] = (
                        ids_v[pl.ds(co + s * SUB + v * LANES, LANES)] + offs
                    )
            for s in range(NSUB):
                pltpu.async_copy(
                    table_hbm.at[idx_v.at[b, s]],
                    rows_v.at[b, pl.ds(s * SUB, SUB)],
                    gsems[b],
                )

        def drain(b):
            for s in range(NSUB):
                pltpu.make_async_copy(
                    table_hbm.at[idx_v.at[b, s]],
                    rows_v.at[b, pl.ds(s * SUB, SUB)],
                    gsems[b],
                ).wait()

        def sum_store(c, b, first):
            if not first:
                pltpu.make_async_copy(
                    out_v.at[b],
                    out_hbm.at[pl.ds(out_base, out_chunk)],
                    osems[b],
                ).wait()

            def body(r, _):
                ro = pl.multiple_of(r * num_tables, num_tables)
                for dd in range(d // LANES):
                    sl = pl.ds(dd * LANES, LANES)
                    acc = rows_v[b, ro, sl]
                    for i in range(1, num_tables):
                        acc = acc + rows_v[b, ro + i, sl]
                    out_v[b, r, sl] = acc
                return 0

            lax.fori_loop(0, out_chunk, body, 0)
            pltpu.async_copy(
                out_v.at[b],
                out_hbm.at[pl.ds(out_base + c * out_chunk, out_chunk)],
                osems[b],
            )

        fire(0, 0)
        fire(1, 1)

        def step(h, _):
            c = pl.multiple_of(h * 2, 2)
            drain(0)
            pl.when(c + 2 < n_chunks)(lambda: fire(c + 2, 0))
            sum_store(c, 0, first=False)
            drain(1)
            pl.when(c + 3 < n_chunks)(lambda: fire(c + 3, 1))
            sum_store(c + 1, 1, first=False)
            return 0

        drain(0)
        fire(2, 0)
        sum_store(0, 0, first=True)
        drain(1)
        fire(3, 1)
        sum_store(1, 1, first=True)
        lax.fori_loop(1, n_chunks // 2, step, 0)

        for b in range(2):
            pltpu.make_async_copy(
                out_v.at[b],
                out_hbm.at[pl.ds(out_base, out_chunk)],
                osems[b],
            ).wait()

    return k


def kernel(input_ids, tables):
    num_tables, vocab, d = tables.shape
    b, s = input_ids.shape
    ids_flat = input_ids.reshape(-1)
    table_flat = tables.reshape(num_tables * vocab, d)
    out = _build(ids_flat.size, num_tables, vocab, d)(ids_flat, table_flat)
    return out.reshape(b, s // num_tables, d)


# speedup vs baseline: 7.4411x; 7.4411x over previous
import functools

import jax
import jax.numpy as jnp
from jax import lax
from jax.experimental import pallas as pl
from jax.experimental.pallas import tpu as pltpu
from jax.experimental.pallas import tpu_sc as plsc

LANES = 16
SUB = 128
NSUB = 4
CHUNK = SUB * NSUB


@functools.cache
def _build(n_total, num_tables, vocab, d):
    info = plsc.get_sparse_core_info()
    nc, ns = info.num_cores, info.num_subcores
    nw = nc * ns
    per_tile = n_total // nw
    n_chunks = per_tile // CHUNK
    out_chunk = CHUNK // num_tables
    out_per_tile = per_tile // num_tables
    n_out = n_total // num_tables
    assert n_total % (nw * CHUNK) == 0 and CHUNK % num_tables == 0
    assert d % LANES == 0 and LANES % num_tables == 0 and n_chunks % 2 == 0

    mesh = plsc.VectorSubcoreMesh(core_axis_name="c", subcore_axis_name="s")

    @functools.partial(
        pl.kernel,
        mesh=mesh,
        compiler_params=pltpu.CompilerParams(use_tc_tiling_on_sc=False),
        out_type=jax.ShapeDtypeStruct((n_out, d), jnp.float32),
        scratch_types=[
            pltpu.VMEM((per_tile,), jnp.int32),
            pltpu.VMEM((2, NSUB, SUB), jnp.int32),
            pltpu.VMEM((2, CHUNK, d), jnp.float32),
            pltpu.VMEM((2, out_chunk, d), jnp.float32),
            pltpu.SemaphoreType.DMA,
            pltpu.SemaphoreType.DMA,
            pltpu.SemaphoreType.DMA,
            pltpu.SemaphoreType.DMA,
        ],
    )
    def k(ids_hbm, table_hbm, out_hbm, ids_v, idx_v, rows_v, out_v,
          gsem0, gsem1, osem0, osem1):
        wid = lax.axis_index("s") * nc + lax.axis_index("c")
        base = wid * per_tile
        out_base = wid * out_per_tile
        gsems = (gsem0, gsem1)
        osems = (osem0, osem1)

        pltpu.sync_copy(ids_hbm.at[pl.ds(base, per_tile)], ids_v)
        offs = (lax.iota(jnp.int32, LANES) % num_tables) * vocab

        def fire(c, b):
            co = pl.multiple_of(c * CHUNK, CHUNK)
            for s in range(NSUB):
                for v in range(SUB // LANES):
                    idx_v[b, s, pl.ds(v * LANES, LANES)] = (
                        ids_v[pl.ds(co + s * SUB + v * LANES, LANES)] + offs
                    )
            for s in range(NSUB):
                pltpu.async_copy(
                    table_hbm.at[idx_v.at[b, s]],
                    rows_v.at[b, pl.ds(s * SUB, SUB)],
                    gsems[b],
                )

        def drain(b):
            for s in range(NSUB):
                pltpu.make_async_copy(
                    table_hbm.at[idx_v.at[b, s]],
                    rows_v.at[b, pl.ds(s * SUB, SUB)],
                    gsems[b],
                ).wait()

        def sum_store(c, b, first):
            if not first:
                pltpu.make_async_copy(
                    out_v.at[b],
                    out_hbm.at[pl.ds(out_base, out_chunk)],
                    osems[b],
                ).wait()

            def body(r, _):
                ro = pl.multiple_of(r * num_tables, num_tables)
                for dd in range(d // LANES):
                    sl = pl.ds(dd * LANES, LANES)
                    acc = rows_v[b, ro, sl]
                    for i in range(1, num_tables):
                        acc = acc + rows_v[b, ro + i, sl]
                    out_v[b, r, sl] = acc
                return 0

            lax.fori_loop(0, out_chunk, body, 0)
            pltpu.async_copy(
                out_v.at[b],
                out_hbm.at[pl.ds(out_base + c * out_chunk, out_chunk)],
                osems[b],
            )

        fire(0, 0)
        fire(1, 1)

        def step(h, _):
            c = pl.multiple_of(h * 2, 2)
            drain(0)
            sum_store(c, 0, first=False)
            pl.when(c + 2 < n_chunks)(lambda: fire(c + 2, 0))
            drain(1)
            sum_store(c + 1, 1, first=False)
            pl.when(c + 3 < n_chunks)(lambda: fire(c + 3, 1))
            return 0

        drain(0)
        sum_store(0, 0, first=True)
        fire(2, 0)
        drain(1)
        sum_store(1, 1, first=True)
        fire(3, 1)
        lax.fori_loop(1, n_chunks // 2, step, 0)

        for b in range(2):
            pltpu.make_async_copy(
                out_v.at[b],
                out_hbm.at[pl.ds(out_base, out_chunk)],
                osems[b],
            ).wait()

    return k


def kernel(input_ids, tables):
    num_tables, vocab, d = tables.shape
    b, s = input_ids.shape
    ids_flat = input_ids.reshape(-1)
    table_flat = tables.reshape(num_tables * vocab, d)
    out = _build(ids_flat.size, num_tables, vocab, d)(ids_flat, table_flat)
    return out.reshape(b, s // num_tables, d)
